# baseline (speedup 1.0000x reference)
import functools

import jax
import jax.numpy as jnp
from jax import lax
from jax.experimental import pallas as pl
from jax.experimental.pallas import tpu as pltpu
from jax.experimental.pallas import tpu_sc as plsc

ROWS = 1024
IN_C = 32 + 1024 + 32768
OUT_C = 1 + IN_C
LEVELS = ((0, 32), (32, 1024), (1056, 32768))
NC = 2
NS = 16
NW = NC * NS
RPW = ROWS // NW
L = 16

_LN2 = 0.6931471805599453


def _vlog(s):
    bits = lax.bitcast_convert_type(s, jnp.int32)
    e = jnp.float32(1.0) * (lax.shift_right_logical(bits, 23) - 127)
    m = lax.bitcast_convert_type(
        (bits & jnp.int32(0x007FFFFF)) | jnp.int32(0x3F800000), jnp.float32)
    t = (m - 1.0) / (m + 1.0)
    t2 = t * t
    p = 1.0 + t2 * (jnp.float32(1 / 3) + t2 * (jnp.float32(1 / 5)
        + t2 * (jnp.float32(1 / 7) + t2 * jnp.float32(1 / 9))))
    return e * jnp.float32(_LN2) + 2.0 * t * p


def _lane_allreduce(v, op):
    lane = lax.iota(jnp.int32, L)
    for sh in (8, 4, 2, 1):
        v = op(v, v.at[lane ^ sh].get(mode="promise_in_bounds"))
    return v


def _row_reduce(inbuf):
    logzs = []
    for start, size in LEVELS:
        nchunks = size // L
        u = min(8, nchunks)

        @plsc.parallel_loop(0, nchunks, step=u,
                            carry=tuple(jnp.full((L,), -jnp.inf, jnp.float32)
                                        for _ in range(u)))
        def _max_body(j, accs, start=start, u=u):
            return tuple(
                jnp.maximum(accs[i], inbuf[pl.ds(start + (j + i) * L, L)])
                for i in range(u))
        maxv = _max_body[0]
        for i in range(1, u):
            maxv = jnp.maximum(maxv, _max_body[i])
        m = _lane_allreduce(maxv, jnp.maximum)

        @plsc.parallel_loop(0, nchunks, step=u,
                            carry=tuple(jnp.zeros((L,), jnp.float32)
                                        for _ in range(u)))
        def _sum_body(j, accs, start=start, u=u, m=m):
            return tuple(
                accs[i] + jnp.exp(inbuf[pl.ds(start + (j + i) * L, L)] - m)
                for i in range(u))
        sumv = _sum_body[0]
        for i in range(1, u):
            sumv = sumv + _sum_body[i]
        s = _lane_allreduce(sumv, jnp.add)

        logzs.append(m + _vlog(s))
    return logzs


def _shift_range(inbuf, outbuf, k0, k1, logz, u):
    assert (k1 - k0) % u == 0

    @plsc.parallel_loop(0, k1 - k0, step=u)
    def _body(j):
        for i in range(u):
            k = k0 + j + i
            outbuf[pl.ds(k * L, L)] = inbuf[pl.ds(k * L - 1, L)] - logz


def _row_shift(inbuf, outbuf, logzs, lane, zero_tail):
    lz0, lz1, lz2 = logzs

    v0 = inbuf[pl.ds(0, L)] - lz0
    rot = v0.at[(lane + L - 1) & (L - 1)].get(mode="promise_in_bounds")
    outbuf[pl.ds(0, L)] = jnp.where(lane == 0, jnp.float32(0.0), rot)

    outbuf[pl.ds(L, L)] = inbuf[pl.ds(L - 1, L)] - lz0
    outbuf[pl.ds(2 * L, L)] = (inbuf[pl.ds(2 * L - 1, L)]
                               - jnp.where(lane == 0, lz0, lz1))
    _shift_range(inbuf, outbuf, 3, 66, lz1, 7)
    outbuf[pl.ds(66 * L, L)] = (inbuf[pl.ds(66 * L - 1, L)]
                                - jnp.where(lane == 0, lz1, lz2))
    _shift_range(inbuf, outbuf, 67, 2114, lz2, 23)

    vl = inbuf[pl.ds(IN_C - L, L)] - lz2
    tail = vl.at[jnp.full((L,), L - 1, jnp.int32)].get(
        mode="promise_in_bounds")
    outbuf[pl.ds(zero_tail + IN_C, L)] = tail


def _levelwise_body(scores_hbm, out_hbm, in0, in1, outbuf, guard,
                    sl0, sl1, ss, rpw=RPW):
    wid = lax.axis_index("s") * NC + lax.axis_index("c")
    row0 = wid * rpw
    lane = lax.iota(jnp.int32, L)
    inbufs = (in0, in1)
    lsems = (sl0, sl1)
    zero_tail = pl.multiple_of(lax.div(wid, jnp.int32(1 << 20)), 8)

    def start_load(b, r):
        pltpu.make_async_copy(
            scores_hbm.at[row0 + r], inbufs[b], lsems[b]).start()

    def wait_load(b):
        pltpu.make_async_copy(
            scores_hbm.at[row0], inbufs[b], lsems[b]).wait()

    def start_store(r):
        pltpu.make_async_copy(outbuf, out_hbm.at[row0 + r], ss).start()

    def wait_store():
        pltpu.make_async_copy(outbuf, out_hbm.at[row0], ss).wait()

    def do_row(r, b, guarded):
        wait_load(b)
        logzs = _row_reduce(inbufs[b])
        if guarded:
            @pl.when(r >= 1)
            def _():
                wait_store()
        else:
            wait_store()
        _row_shift(inbufs[b], outbuf, logzs, lane, zero_tail)
        start_store(r)

    start_load(0, 0)
    start_load(1, 1)

    @pl.loop(0, rpw - 2, step=2)
    def _grp(base):
        for b in range(2):
            r = base + b
            do_row(r, b, b == 0)
            start_load(b, r + 2)

    for r in (rpw - 2, rpw - 1):
        do_row(r, r % 2, False)
    wait_store()


def _make_half(rows):
    mesh = plsc.VectorSubcoreMesh(core_axis_name="c", subcore_axis_name="s")
    body = functools.partial(_levelwise_body, rpw=rows // NW)
    return functools.partial(
        pl.kernel,
        mesh=mesh,
        out_type=jax.ShapeDtypeStruct((rows, OUT_C), jnp.float32),
        scratch_types=[
            pltpu.VMEM((IN_C,), jnp.float32),
            pltpu.VMEM((IN_C,), jnp.float32),
            pltpu.VMEM((OUT_C,), jnp.float32),
            pltpu.VMEM((L,), jnp.float32),
            pltpu.SemaphoreType.DMA,
            pltpu.SemaphoreType.DMA,
            pltpu.SemaphoreType.DMA,
        ],
    )(body)


@jax.jit
def kernel(scores):
    half = ROWS // 2
    f = _make_half(half)
    top = f(scores[:half])
    bot = f(scores[half:])
    return jnp.concatenate([top, bot], axis=0)


# speedup vs baseline: 1.2900x; 1.2900x over previous
import functools

import jax
import jax.numpy as jnp
from jax import lax
from jax.experimental import pallas as pl
from jax.experimental.pallas import tpu as pltpu
from jax.experimental.pallas import tpu_sc as plsc

ROWS = 1024
IN_C = 32 + 1024 + 32768
OUT_C = 1 + IN_C
LEVELS = ((0, 32), (32, 1024), (1056, 32768))
NC = 2
NS = 16
NW = NC * NS
RPW = ROWS // NW
L = 16

_LN2 = 0.6931471805599453


def _vlog(s):
    bits = lax.bitcast_convert_type(s, jnp.int32)
    e = jnp.float32(1.0) * (lax.shift_right_logical(bits, 23) - 127)
    m = lax.bitcast_convert_type(
        (bits & jnp.int32(0x007FFFFF)) | jnp.int32(0x3F800000), jnp.float32)
    t = (m - 1.0) / (m + 1.0)
    t2 = t * t
    p = 1.0 + t2 * (jnp.float32(1 / 3) + t2 * (jnp.float32(1 / 5)
        + t2 * (jnp.float32(1 / 7) + t2 * jnp.float32(1 / 9))))
    return e * jnp.float32(_LN2) + 2.0 * t * p


def _lane_allreduce(v, op):
    lane = lax.iota(jnp.int32, L)
    for sh in (8, 4, 2, 1):
        v = op(v, v.at[lane ^ sh].get(mode="promise_in_bounds"))
    return v


def _row_reduce(inbuf):
    logzs = []
    for start, size in LEVELS:
        nchunks = size // L
        u = min(8, nchunks)

        @plsc.parallel_loop(0, nchunks, step=u,
                            carry=tuple(jnp.full((L,), -jnp.inf, jnp.float32)
                                        for _ in range(u)))
        def _max_body(j, accs, start=start, u=u):
            return tuple(
                jnp.maximum(accs[i], inbuf[pl.ds(start + (j + i) * L, L)])
                for i in range(u))
        maxv = _max_body[0]
        for i in range(1, u):
            maxv = jnp.maximum(maxv, _max_body[i])
        m = _lane_allreduce(maxv, jnp.maximum)

        @plsc.parallel_loop(0, nchunks, step=u,
                            carry=tuple(jnp.zeros((L,), jnp.float32)
                                        for _ in range(u)))
        def _sum_body(j, accs, start=start, u=u, m=m):
            return tuple(
                accs[i] + jnp.exp(inbuf[pl.ds(start + (j + i) * L, L)] - m)
                for i in range(u))
        sumv = _sum_body[0]
        for i in range(1, u):
            sumv = sumv + _sum_body[i]
        s = _lane_allreduce(sumv, jnp.add)

        logzs.append(m + _vlog(s))
    return logzs


def _shift_range(inbuf, outbuf, k0, k1, logz, u):
    assert (k1 - k0) % u == 0

    @plsc.parallel_loop(0, k1 - k0, step=u)
    def _body(j):
        for i in range(u):
            k = k0 + j + i
            outbuf[pl.ds(k * L, L)] = inbuf[pl.ds(k * L - 1, L)] - logz


def _row_shift(inbuf, outbuf, logzs, lane, zero_tail):
    lz0, lz1, lz2 = logzs

    v0 = inbuf[pl.ds(0, L)] - lz0
    rot = v0.at[(lane + L - 1) & (L - 1)].get(mode="promise_in_bounds")
    outbuf[pl.ds(0, L)] = jnp.where(lane == 0, jnp.float32(0.0), rot)

    outbuf[pl.ds(L, L)] = inbuf[pl.ds(L - 1, L)] - lz0
    outbuf[pl.ds(2 * L, L)] = (inbuf[pl.ds(2 * L - 1, L)]
                               - jnp.where(lane == 0, lz0, lz1))
    _shift_range(inbuf, outbuf, 3, 66, lz1, 7)
    outbuf[pl.ds(66 * L, L)] = (inbuf[pl.ds(66 * L - 1, L)]
                                - jnp.where(lane == 0, lz1, lz2))
    _shift_range(inbuf, outbuf, 67, 2114, lz2, 23)

    vl = inbuf[pl.ds(IN_C - L, L)] - lz2
    tail = vl.at[jnp.full((L,), L - 1, jnp.int32)].get(
        mode="promise_in_bounds")
    outbuf[pl.ds(zero_tail + IN_C, L)] = tail


def _levelwise_body(scores_hbm, out_hbm, in0, in1, outbuf, guard,
                    sl0, sl1, ss):
    wid = lax.axis_index("s") * NC + lax.axis_index("c")
    row0 = wid * RPW
    lane = lax.iota(jnp.int32, L)
    inbufs = (in0, in1)
    lsems = (sl0, sl1)
    zero_tail = pl.multiple_of(lax.div(wid, jnp.int32(1 << 20)), 8)

    def start_load(b, r):
        pltpu.make_async_copy(
            scores_hbm.at[row0 + r], inbufs[b], lsems[b]).start()

    def wait_load(b):
        pltpu.make_async_copy(
            scores_hbm.at[row0], inbufs[b], lsems[b]).wait()

    def start_store(r):
        pltpu.make_async_copy(outbuf, out_hbm.at[row0 + r], ss).start()

    def wait_store():
        pltpu.make_async_copy(outbuf, out_hbm.at[row0], ss).wait()

    def do_row(r, b, guarded):
        wait_load(b)
        logzs = _row_reduce(inbufs[b])
        if guarded:
            @pl.when(r >= 1)
            def _():
                wait_store()
        else:
            wait_store()
        _row_shift(inbufs[b], outbuf, logzs, lane, zero_tail)
        start_store(r)

    start_load(0, 0)
    start_load(1, 1)

    @pl.loop(0, RPW - 2, step=2)
    def _grp(base):
        for b in range(2):
            r = base + b
            do_row(r, b, b == 0)
            start_load(b, r + 2)

    for r in (RPW - 2, RPW - 1):
        do_row(r, r % 2, False)
    wait_store()


@jax.jit
def kernel(scores):
    mesh = plsc.VectorSubcoreMesh(core_axis_name="c", subcore_axis_name="s")
    f = functools.partial(
        pl.kernel,
        mesh=mesh,
        out_type=jax.ShapeDtypeStruct((ROWS, OUT_C), jnp.float32),
        scratch_types=[
            pltpu.VMEM((IN_C,), jnp.float32),
            pltpu.VMEM((IN_C,), jnp.float32),
            pltpu.VMEM((OUT_C,), jnp.float32),
            pltpu.VMEM((L,), jnp.float32),
            pltpu.SemaphoreType.DMA,
            pltpu.SemaphoreType.DMA,
            pltpu.SemaphoreType.DMA,
        ],
    )(_levelwise_body)
    return f(scores)
